# Initial kernel scaffold; baseline (speedup 1.0000x reference)
import jax
import jax.numpy as jnp
from jax.experimental import pallas as pl


def kernel(x, W_ih, W_hh, b_ih, b_hh):
    raise NotImplementedError("write your pallas kernel here")


# speedup vs baseline: 2.5685x; 2.5685x over previous
import functools

import jax
import jax.numpy as jnp
from jax.experimental import pallas as pl
from jax.experimental.pallas import tpu as pltpu

_B, _T, _D, _H = 1024, 200, 64, 64
_STEPS_PER_BLOCK = 2


def _lstm_body(x_ref, w_ref, b_ref, out_ref, h_ref, c_ref):
    t = pl.program_id(0)

    @pl.when(t == 0)
    def _init():
        h_ref[...] = jnp.zeros_like(h_ref)
        c_ref[...] = jnp.zeros_like(c_ref)

    w = w_ref[...]
    b = b_ref[0:1, :]

    def step(xt, h, c):
        xh = jnp.concatenate([xt, h], axis=-1)
        gates = jax.lax.dot_general(
            xh, w,
            dimension_numbers=(((1,), (0,)), ((), ())),
            preferred_element_type=jnp.float32,
        ) + b
        i_g = jax.nn.sigmoid(gates[:, 0 * _H:1 * _H])
        f_g = jax.nn.sigmoid(gates[:, 1 * _H:2 * _H])
        g_g = jnp.tanh(gates[:, 2 * _H:3 * _H])
        o_g = jax.nn.sigmoid(gates[:, 3 * _H:4 * _H])
        c_new = f_g * c + i_g * g_g
        h_new = o_g * jnp.tanh(c_new)
        return h_new, c_new

    blk = x_ref[...]
    h = h_ref[...]
    c = c_ref[...]
    outs = []
    for s in range(_STEPS_PER_BLOCK):
        h, c = step(blk[:, s * _D:(s + 1) * _D], h, c)
        outs.append(h)
    out_ref[...] = jnp.concatenate(outs, axis=-1)
    h_ref[...] = h
    c_ref[...] = c


@functools.partial(jax.jit, static_argnames=())
def kernel(x, W_ih, W_hh, b_ih, b_hh):
    w_cat = jnp.concatenate([W_ih.T, W_hh.T], axis=0)
    b_row = jnp.broadcast_to((b_ih + b_hh)[None, :], (8, 4 * _H))
    x2 = x.reshape(_B, _T * _D)

    grid = (_T // _STEPS_PER_BLOCK,)
    blk_w = _STEPS_PER_BLOCK * _D

    out = pl.pallas_call(
        _lstm_body,
        grid=grid,
        in_specs=[
            pl.BlockSpec((_B, blk_w), lambda t: (0, t)),
            pl.BlockSpec((_D + _H, 4 * _H), lambda t: (0, 0)),
            pl.BlockSpec((8, 4 * _H), lambda t: (0, 0)),
        ],
        out_specs=pl.BlockSpec((_B, _STEPS_PER_BLOCK * _H), lambda t: (0, t)),
        out_shape=jax.ShapeDtypeStruct((_B, _T * _H), jnp.float32),
        scratch_shapes=[
            pltpu.VMEM((_B, _H), jnp.float32),
            pltpu.VMEM((_B, _H), jnp.float32),
        ],
        compiler_params=pltpu.CompilerParams(
            dimension_semantics=("arbitrary",),
        ),
    )(x2, w_cat, b_row)

    return out.reshape(_B, _T, _H)


# speedup vs baseline: 2.7121x; 1.0559x over previous
import jax
import jax.numpy as jnp
from jax.experimental import pallas as pl
from jax.experimental.pallas import tpu as pltpu

_B, _T, _D, _H = 1024, 200, 64, 64
_S = 8


def _sig(z):
    return jnp.tanh(z * 0.5) * 0.5 + 0.5


def _lstm_body(x_ref, wx_ref, wh_ref, b_ref, out_ref, h_ref, c_ref):
    t = pl.program_id(0)

    @pl.when(t == 0)
    def _init():
        h_ref[...] = jnp.zeros_like(h_ref)
        c_ref[...] = jnp.zeros_like(c_ref)

    wx = wx_ref[...]
    wh = wh_ref[...]
    b = b_ref[0:1, :]
    dn = (((1,), (0,)), ((), ()))

    blk = x_ref[...]
    h = h_ref[...]
    c = c_ref[...]
    outs = []
    for s in range(_S):
        xg = jax.lax.dot_general(
            blk[:, s * _D:(s + 1) * _D], wx, dn,
            preferred_element_type=jnp.float32)
        hg = jax.lax.dot_general(
            h, wh, dn, preferred_element_type=jnp.float32)
        gates = xg + hg + b
        i_g = _sig(gates[:, 0 * _H:1 * _H])
        f_g = _sig(gates[:, 1 * _H:2 * _H])
        g_g = jnp.tanh(gates[:, 2 * _H:3 * _H])
        o_g = _sig(gates[:, 3 * _H:4 * _H])
        c = f_g * c + i_g * g_g
        h = o_g * jnp.tanh(c)
        outs.append(h)
    out_ref[...] = jnp.concatenate(outs, axis=-1)
    h_ref[...] = h
    c_ref[...] = c


def kernel(x, W_ih, W_hh, b_ih, b_hh):
    wx = W_ih.T
    wh = W_hh.T
    b_row = jnp.broadcast_to((b_ih + b_hh)[None, :], (8, 4 * _H))
    x2 = x.reshape(_B, _T * _D)

    grid = (_T // _S,)

    out = pl.pallas_call(
        _lstm_body,
        grid=grid,
        in_specs=[
            pl.BlockSpec((_B, _S * _D), lambda t: (0, t)),
            pl.BlockSpec((_D, 4 * _H), lambda t: (0, 0)),
            pl.BlockSpec((_H, 4 * _H), lambda t: (0, 0)),
            pl.BlockSpec((8, 4 * _H), lambda t: (0, 0)),
        ],
        out_specs=pl.BlockSpec((_B, _S * _H), lambda t: (0, t)),
        out_shape=jax.ShapeDtypeStruct((_B, _T * _H), jnp.float32),
        scratch_shapes=[
            pltpu.VMEM((_B, _H), jnp.float32),
            pltpu.VMEM((_B, _H), jnp.float32),
        ],
        compiler_params=pltpu.CompilerParams(
            dimension_semantics=("arbitrary",),
        ),
    )(x2, wx, wh, b_row)

    return out.reshape(_B, _T, _H)
